# baseline (speedup 1.0000x reference)
import functools

import jax
import jax.numpy as jnp
from jax import lax
from jax.experimental import pallas as pl
from jax.experimental.pallas import tpu as pltpu
from jax.experimental.pallas import tpu_sc as plsc

_NW = 32
_K = 72
_F0 = 0.72


def _sc_segment_sum(h, src3, dst3, zeros_tile, n_pad, c0, c1):
  n, d = h.shape
  rows_z = n_pad // 16
  mesh = plsc.VectorSubcoreMesh(core_axis_name="c", subcore_axis_name="s")

  @functools.partial(
      pl.kernel,
      out_type=jax.ShapeDtypeStruct((2, n_pad, d), jnp.float32),
      mesh=mesh,
      scratch_types=[
          pltpu.VMEM((3, _K), jnp.int32),
          pltpu.VMEM((3, _K), jnp.int32),
          pltpu.VMEM((3, _K, d), jnp.float32),
          pltpu.VMEM_SHARED((n_pad, d), jnp.float32),
          pltpu.SemaphoreType.DMA,
          pltpu.SemaphoreType.DMA,
          pltpu.SemaphoreType.DMA,
          pltpu.SemaphoreType.DMA,
          pltpu.SemaphoreType.DMA,
          pltpu.SemaphoreType.DMA,
          pltpu.SemaphoreType.DMA,
          pltpu.SemaphoreType.DMA,
          pltpu.SemaphoreType.DMA,
      ],
  )
  def sc_kernel(h_hbm, src_hbm, dst_hbm, z_hbm, out_hbm,
                src_i, dst_i, rows_v, acc_sh,
                si0, si1, si2, sr0, sr1, sr2, ss0, ss1, ss2):
    cid = lax.axis_index("c")
    sid = lax.axis_index("s")
    my_c = jnp.where(cid == 0, c0, c1)
    semi = (si0, si1, si2)
    semr = (sr0, sr1, sr2)
    sems = (ss0, ss1, ss2)

    def start_idx(jj, b):
      pltpu.async_copy(src_hbm.at[cid].at[sid].at[jj], src_i.at[b], semi[b])
      pltpu.async_copy(dst_hbm.at[cid].at[sid].at[jj], dst_i.at[b], semi[b])

    def wait_idx(jj, b):
      pltpu.make_async_copy(src_hbm.at[cid].at[sid].at[jj], src_i.at[b],
                            semi[b]).wait()
      pltpu.make_async_copy(dst_hbm.at[cid].at[sid].at[jj], dst_i.at[b],
                            semi[b]).wait()

    def start_gather(b):
      pltpu.async_copy(h_hbm.at[src_i.at[b]], rows_v.at[b], semr[b])

    def wait_gather(b):
      pltpu.make_async_copy(h_hbm.at[src_i.at[b]], rows_v.at[b],
                            semr[b]).wait()

    def start_scatter(b):
      pltpu.async_copy(rows_v.at[b], acc_sh.at[dst_i.at[b]], sems[b],
                       add=True)

    def drain_scatter(b):
      pltpu.make_async_copy(rows_v.at[b], acc_sh.at[dst_i.at[b]],
                            sems[b]).wait()

    pltpu.sync_copy(z_hbm, acc_sh.at[pl.ds(sid * rows_z, rows_z)])
    start_idx(0, 0)
    start_idx(1, 1)
    wait_idx(0, 0)
    start_gather(0)
    plsc.subcore_barrier()

    def body(i, carry):
      for b in range(3):
        jj = i * 3 + b

        @pl.when(jj >= 1)
        def _():
          drain_scatter((b + 2) % 3)

        @pl.when(jj + 2 < my_c)
        def _():
          start_idx(jj + 2, (b + 2) % 3)

        @pl.when(jj + 1 < my_c)
        def _():
          wait_idx(jj + 1, (b + 1) % 3)
          start_gather((b + 1) % 3)

        wait_gather(b)
        start_scatter(b)
      return carry

    lax.fori_loop(0, my_c // 3, body, 0)
    drain_scatter(2)
    plsc.subcore_barrier()
    pltpu.sync_copy(acc_sh.at[pl.ds(sid * rows_z, rows_z)],
                    out_hbm.at[cid].at[pl.ds(sid * rows_z, rows_z)])

  return sc_kernel(h, src3, dst3, zeros_tile)


def _dense_layer(h, parts, w1, b1, g1, bt1, w2, b2, bng, bnb, relu_out):
  n, d = h.shape
  eps = 1e-5

  def body(h_ref, p_ref, w1_ref, b1_ref, g1_ref, bt1_ref,
           w2_ref, b2_ref, bng_ref, bnb_ref, o_ref):
    z = h_ref[...] + p_ref[0, :h_ref.shape[0]] + p_ref[1, :h_ref.shape[0]]
    z = jnp.dot(z, w1_ref[...], preferred_element_type=jnp.float32)
    z = z + b1_ref[...]
    mu = jnp.mean(z, axis=0, keepdims=True)
    var = jnp.mean((z - mu) ** 2, axis=0, keepdims=True)
    z = (z - mu) * lax.rsqrt(var + eps) * g1_ref[...] + bt1_ref[...]
    z = jnp.maximum(z, 0.0)
    z = jnp.dot(z, w2_ref[...], preferred_element_type=jnp.float32)
    z = z + b2_ref[...]
    mu2 = jnp.mean(z, axis=0, keepdims=True)
    var2 = jnp.mean((z - mu2) ** 2, axis=0, keepdims=True)
    z = (z - mu2) * lax.rsqrt(var2 + eps) * bng_ref[...] + bnb_ref[...]
    if relu_out:
      z = jnp.maximum(z, 0.0)
    o_ref[...] = z

  return pl.pallas_call(
      body,
      out_shape=jax.ShapeDtypeStruct((n, d), jnp.float32),
  )(h, parts, w1, b1.reshape(1, d), g1.reshape(1, d), bt1.reshape(1, d),
    w2, b2.reshape(1, d), bng.reshape(1, d), bnb.reshape(1, d))


def kernel(x, edge_index, W1, b1, g1, bt1, W2, b2, bng, bnb):
  n, d = x.shape
  e = edge_index.shape[1]
  num_layers = W1.shape[0]

  c0 = max(3, int(round(e * _F0 / (16 * _K * 3))) * 3)
  e0 = min(e, 16 * c0 * _K)
  e1 = e - e0
  c1 = -(-e1 // (16 * _K))
  c1 += (-c1) % 3
  cmax = max(c0, c1)

  def _side(srcs, dsts, cc):
    pad = 16 * cc * _K - srcs.shape[0]
    s = jnp.concatenate([srcs, jnp.zeros((pad,), jnp.int32)]).reshape(
        16, cc, _K)
    t = jnp.concatenate([dsts, jnp.full((pad,), n, jnp.int32)]).reshape(
        16, cc, _K)
    if cc < cmax:
      s = jnp.pad(s, ((0, 0), (0, cmax - cc), (0, 0)))
      t = jnp.pad(t, ((0, 0), (0, cmax - cc), (0, 0)), constant_values=n)
    return s, t

  s0, t0 = _side(edge_index[0, :e0], edge_index[1, :e0], c0)
  s1, t1 = _side(edge_index[0, e0:], edge_index[1, e0:], c1)
  src4 = jnp.stack([s0, s1])
  dst4 = jnp.stack([t0, t1])

  n_pad = ((n // 128) + 1) * 128
  zeros_tile = jnp.zeros((n_pad // 16, d), jnp.float32)

  h = x
  for l in range(num_layers):
    parts = _sc_segment_sum(h, src4, dst4, zeros_tile, n_pad, c0, c1)
    h = _dense_layer(h, parts, W1[l], b1[l], g1[l], bt1[l],
                     W2[l], b2[l], bng[l], bnb[l], relu_out=(l < num_layers - 1))
  return h


# speedup vs baseline: 1.1103x; 1.1103x over previous
import functools

import jax
import jax.numpy as jnp
from jax import lax
from jax.experimental import pallas as pl
from jax.experimental.pallas import tpu as pltpu
from jax.experimental.pallas import tpu_sc as plsc

_NW = 32
_K = 72
_F0 = 0.60


def _sc_segment_sum(h, src3, dst3, zeros_tile, n_pad, c0, c1):
  n, d = h.shape
  rows_z = n_pad // 16
  mesh = plsc.VectorSubcoreMesh(core_axis_name="c", subcore_axis_name="s")

  @functools.partial(
      pl.kernel,
      out_type=jax.ShapeDtypeStruct((2, n_pad, d), jnp.float32),
      mesh=mesh,
      scratch_types=[
          pltpu.VMEM((3, _K), jnp.int32),
          pltpu.VMEM((3, _K), jnp.int32),
          pltpu.VMEM((3, _K, d), jnp.float32),
          pltpu.VMEM_SHARED((n_pad, d), jnp.float32),
          pltpu.SemaphoreType.DMA,
          pltpu.SemaphoreType.DMA,
          pltpu.SemaphoreType.DMA,
          pltpu.SemaphoreType.DMA,
          pltpu.SemaphoreType.DMA,
          pltpu.SemaphoreType.DMA,
          pltpu.SemaphoreType.DMA,
          pltpu.SemaphoreType.DMA,
          pltpu.SemaphoreType.DMA,
      ],
  )
  def sc_kernel(h_hbm, src_hbm, dst_hbm, z_hbm, out_hbm,
                src_i, dst_i, rows_v, acc_sh,
                si0, si1, si2, sr0, sr1, sr2, ss0, ss1, ss2):
    cid = lax.axis_index("c")
    sid = lax.axis_index("s")
    my_c = jnp.where(cid == 0, c0, c1)
    semi = (si0, si1, si2)
    semr = (sr0, sr1, sr2)
    sems = (ss0, ss1, ss2)

    def start_idx(jj, b):
      pltpu.async_copy(src_hbm.at[cid].at[sid].at[jj], src_i.at[b], semi[b])
      pltpu.async_copy(dst_hbm.at[cid].at[sid].at[jj], dst_i.at[b], semi[b])

    def wait_idx(jj, b):
      pltpu.make_async_copy(src_hbm.at[cid].at[sid].at[jj], src_i.at[b],
                            semi[b]).wait()
      pltpu.make_async_copy(dst_hbm.at[cid].at[sid].at[jj], dst_i.at[b],
                            semi[b]).wait()

    def start_gather(b):
      pltpu.async_copy(h_hbm.at[src_i.at[b]], rows_v.at[b], semr[b])

    def wait_gather(b):
      pltpu.make_async_copy(h_hbm.at[src_i.at[b]], rows_v.at[b],
                            semr[b]).wait()

    def start_scatter(b):
      pltpu.async_copy(rows_v.at[b], acc_sh.at[dst_i.at[b]], sems[b],
                       add=True)

    def drain_scatter(b):
      pltpu.make_async_copy(rows_v.at[b], acc_sh.at[dst_i.at[b]],
                            sems[b]).wait()

    pltpu.sync_copy(z_hbm, acc_sh.at[pl.ds(sid * rows_z, rows_z)])
    start_idx(0, 0)
    start_idx(1, 1)
    wait_idx(0, 0)
    start_gather(0)
    plsc.subcore_barrier()

    def body(i, carry):
      for b in range(3):
        jj = i * 3 + b

        @pl.when(jj >= 1)
        def _():
          drain_scatter((b + 2) % 3)

        @pl.when(jj + 2 < my_c)
        def _():
          start_idx(jj + 2, (b + 2) % 3)

        @pl.when(jj + 1 < my_c)
        def _():
          wait_idx(jj + 1, (b + 1) % 3)
          start_gather((b + 1) % 3)

        wait_gather(b)
        start_scatter(b)
      return carry

    lax.fori_loop(0, my_c // 3, body, 0)
    drain_scatter(2)
    plsc.subcore_barrier()
    pltpu.sync_copy(acc_sh.at[pl.ds(sid * rows_z, rows_z)],
                    out_hbm.at[cid].at[pl.ds(sid * rows_z, rows_z)])

  return sc_kernel(h, src3, dst3, zeros_tile)


def _dense_layer(h, parts, w1, b1, g1, bt1, w2, b2, bng, bnb, relu_out):
  n, d = h.shape
  eps = 1e-5

  def body(h_ref, p_ref, w1_ref, b1_ref, g1_ref, bt1_ref,
           w2_ref, b2_ref, bng_ref, bnb_ref, o_ref):
    z = h_ref[...] + p_ref[0, :h_ref.shape[0]] + p_ref[1, :h_ref.shape[0]]
    z = jnp.dot(z, w1_ref[...], preferred_element_type=jnp.float32)
    z = z + b1_ref[...]
    mu = jnp.mean(z, axis=0, keepdims=True)
    var = jnp.mean((z - mu) ** 2, axis=0, keepdims=True)
    z = (z - mu) * lax.rsqrt(var + eps) * g1_ref[...] + bt1_ref[...]
    z = jnp.maximum(z, 0.0)
    z = jnp.dot(z, w2_ref[...], preferred_element_type=jnp.float32)
    z = z + b2_ref[...]
    mu2 = jnp.mean(z, axis=0, keepdims=True)
    var2 = jnp.mean((z - mu2) ** 2, axis=0, keepdims=True)
    z = (z - mu2) * lax.rsqrt(var2 + eps) * bng_ref[...] + bnb_ref[...]
    if relu_out:
      z = jnp.maximum(z, 0.0)
    o_ref[...] = z

  return pl.pallas_call(
      body,
      out_shape=jax.ShapeDtypeStruct((n, d), jnp.float32),
  )(h, parts, w1, b1.reshape(1, d), g1.reshape(1, d), bt1.reshape(1, d),
    w2, b2.reshape(1, d), bng.reshape(1, d), bnb.reshape(1, d))


def kernel(x, edge_index, W1, b1, g1, bt1, W2, b2, bng, bnb):
  n, d = x.shape
  e = edge_index.shape[1]
  num_layers = W1.shape[0]

  c0 = max(3, int(round(e * _F0 / (16 * _K * 3))) * 3)
  e0 = min(e, 16 * c0 * _K)
  e1 = e - e0
  c1 = -(-e1 // (16 * _K))
  c1 += (-c1) % 3
  cmax = max(c0, c1)

  def _side(srcs, dsts, cc):
    pad = 16 * cc * _K - srcs.shape[0]
    s = jnp.concatenate([srcs, jnp.zeros((pad,), jnp.int32)]).reshape(
        16, cc, _K)
    t = jnp.concatenate([dsts, jnp.full((pad,), n, jnp.int32)]).reshape(
        16, cc, _K)
    if cc < cmax:
      s = jnp.pad(s, ((0, 0), (0, cmax - cc), (0, 0)))
      t = jnp.pad(t, ((0, 0), (0, cmax - cc), (0, 0)), constant_values=n)
    return s, t

  s0, t0 = _side(edge_index[0, :e0], edge_index[1, :e0], c0)
  s1, t1 = _side(edge_index[0, e0:], edge_index[1, e0:], c1)
  src4 = jnp.stack([s0, s1])
  dst4 = jnp.stack([t0, t1])

  n_pad = ((n // 128) + 1) * 128
  zeros_tile = jnp.zeros((n_pad // 16, d), jnp.float32)

  h = x
  for l in range(num_layers):
    parts = _sc_segment_sum(h, src4, dst4, zeros_tile, n_pad, c0, c1)
    h = _dense_layer(h, parts, W1[l], b1[l], g1[l], bt1[l],
                     W2[l], b2[l], bng[l], bnb[l], relu_out=(l < num_layers - 1))
  return h
